# Initial kernel scaffold; baseline (speedup 1.0000x reference)
import jax
import jax.numpy as jnp
from jax.experimental import pallas as pl


def kernel(x, embed_site):
    raise NotImplementedError("write your pallas kernel here")


# speedup vs baseline: 6.4818x; 6.4818x over previous
import functools

import jax
import jax.numpy as jnp
from jax import lax
from jax.experimental import pallas as pl
from jax.experimental.pallas import tpu as pltpu
from jax.experimental.pallas import tpu_sc as plsc

VOCAB = 205
D = 16
C = 128
NB = 8
NC, NS = 2, 16
NW = NC * NS


def _build(B):
    assert B % (NW * C) == 0
    nchunks_total = B // C
    nchunks = nchunks_total // NW
    assert nchunks % NB == 0
    nrounds = nchunks // NB

    mesh = plsc.VectorSubcoreMesh(core_axis_name="c", subcore_axis_name="s")

    @functools.partial(
        pl.kernel,
        out_type=jax.ShapeDtypeStruct((B, D), jnp.float32),
        mesh=mesh,
        scratch_types=(
            [pltpu.VMEM((nchunks, C), jnp.int32),
             pltpu.VMEM((NB, C, D), jnp.float32)]
            + [pltpu.SemaphoreType.DMA] * NB
            + [pltpu.SemaphoreType.DMA] * NB
        ),
        compiler_params=pltpu.CompilerParams(use_tc_tiling_on_sc=False),
    )
    def k(x_hbm, table_hbm, out_hbm, idx_v, rows_v, *sems):
        sem_g = sems[:NB]
        sem_w = sems[NB:]
        wid = lax.axis_index("s") * NC + lax.axis_index("c")
        chunk0 = wid * nchunks

        pltpu.sync_copy(x_hbm.at[pl.ds(chunk0, nchunks)], idx_v)

        def gather(g, b):
            return pltpu.make_async_copy(
                table_hbm.at[idx_v.at[g]], rows_v.at[b], sem_g[b])

        def write(g, b):
            return pltpu.make_async_copy(
                rows_v.at[b], out_hbm.at[pl.ds((chunk0 + g) * C, C)], sem_w[b])

        for b in range(NB):
            gather(b, b).start()

        def round_body(r, _):
            for b in range(NB):
                g = r * NB + b
                gather(g, b).wait()
                write(g, b).start()
            for b in range(NB):
                g = r * NB + b
                write(g, b).wait()
                gather(g + NB, b).start()
            return 0

        lax.fori_loop(0, nrounds - 1, round_body, 0)

        r = nrounds - 1
        for b in range(NB):
            g = r * NB + b
            gather(g, b).wait()
            write(g, b).start()
        for b in range(NB):
            g = r * NB + b
            write(g, b).wait()

    return k


def kernel(x, embed_site):
    B = x.size
    x2 = x.reshape(B // C, C).astype(jnp.int32)
    return _build(B)(x2, embed_site)
